# Initial kernel scaffold; baseline (speedup 1.0000x reference)
import jax
import jax.numpy as jnp
from jax.experimental import pallas as pl


def kernel(x, pos, edge_index, W_lin_0, W_src_0, W_dst_0, W_pos_0, b_pos_0, W_lin_1, W_src_1, W_dst_1, W_pos_1, b_pos_1):
    raise NotImplementedError("write your pallas kernel here")


# speedup vs baseline: 7.7808x; 7.7808x over previous
import functools
import jax
import jax.numpy as jnp
from jax import lax
from jax.experimental import pallas as pl
from jax.experimental.pallas import tpu as pltpu
from jax.experimental.pallas import tpu_sc as plsc

_N = 10000
_C = 128
_K = 16
_E = _N * _K
_NTILE = 16
_CH = 80
_RPT = _N // _NTILE
_EPT = _E // _NTILE
_BLK = 1000


def _prologue_body(x_ref, pos_ref, ws_ref, wd_ref, wl_ref, wp_ref, b_ref,
                   allpg_ref, ed_ref, t_ref):
    x = x_ref[...]
    p = pos_ref[...]
    for d in range(2):
        q = jnp.dot(p, wp_ref[d], preferred_element_type=jnp.float32)
        b = b_ref[d, 0:1, :]
        t = q + b
        s = jnp.dot(x, ws_ref[d], preferred_element_type=jnp.float32) + q
        dp = jnp.dot(x, wd_ref[d], preferred_element_type=jnp.float32) + t
        v = jnp.dot(x, wl_ref[d], preferred_element_type=jnp.float32) - q
        pe = jnp.exp(-s)
        g = pe * v
        ed_ref[d] = jnp.exp(dp)
        t_ref[d] = t
        for c in range(2):
            allpg_ref[2 * d + c] = jnp.concatenate(
                [pe[:, 64 * c:64 * (c + 1)], g[:, 64 * c:64 * (c + 1)]], axis=1)


def _epilogue_body(acc_ref, ed_ref, t_ref, x_ref, out_ref):
    h = None
    for d in range(2):
        a = jnp.concatenate([acc_ref[2 * d][:, :64], acc_ref[2 * d + 1][:, :64]],
                            axis=1)
        bt = jnp.concatenate([acc_ref[2 * d][:, 64:], acc_ref[2 * d + 1][:, 64:]],
                             axis=1)
        ed = ed_ref[d]
        t = t_ref[d]
        hd = ed * (bt + t * a) / (ed * a + 1e-16)
        h = hd if h is None else jnp.maximum(h, hd)
    out_ref[...] = h + x_ref[...]


_ZCH = 80
_NZC = _N // _ZCH
_NZI = -(-_NZC // _NTILE)


def _sc_body(allpg, srcidx, dstidx, zeros_hbm, out,
             sidx_v, didx_v, rows_v, accum, sem):
    c = lax.axis_index("c")
    s = lax.axis_index("s")
    for d in range(2):
        @pl.loop(0, _NZI)
        def _zero(i):
            t = i * _NTILE + s

            @pl.when(t < _NZC)
            def _():
                pltpu.sync_copy(zeros_hbm, accum.at[pl.ds(t * _ZCH, _ZCH)])

        plsc.subcore_barrier()

        ebase = d * _E + s * _EPT

        @pl.loop(0, _EPT // _CH)
        def _chunk(ti):
            base = ebase + ti * _CH
            pltpu.sync_copy(srcidx.at[pl.ds(c * 2 * _E + base, _CH)], sidx_v)
            pltpu.sync_copy(dstidx.at[pl.ds(base, _CH)], didx_v)
            pltpu.async_copy(allpg.at[sidx_v], rows_v, sem).wait()
            pltpu.sync_copy(rows_v, accum.at[didx_v], add=True)

        plsc.subcore_barrier()

        @pl.loop(0, _NZI)
        def _dump(i):
            t = i * _NTILE + s

            @pl.when(t < _NZC)
            def _():
                pltpu.sync_copy(accum.at[pl.ds(t * _ZCH, _ZCH)],
                                out.at[pl.ds((2 * d + c) * _N + t * _ZCH, _ZCH)])


def kernel(x, pos, edge_index, W_lin_0, W_src_0, W_dst_0, W_pos_0, b_pos_0,
           W_lin_1, W_src_1, W_dst_1, W_pos_1, b_pos_1):
    f32 = jnp.float32
    ws = jnp.stack([W_src_0, W_src_1])
    wd = jnp.stack([W_dst_0, W_dst_1])
    wl = jnp.stack([W_lin_0, W_lin_1])
    wp = jnp.concatenate(
        [jnp.stack([W_pos_0, W_pos_1]), jnp.zeros((2, 5, _C), f32)], axis=1)
    bp = jnp.concatenate(
        [jnp.stack([b_pos_0, b_pos_1])[:, None, :], jnp.zeros((2, 7, _C), f32)],
        axis=1)
    posp = jnp.concatenate([pos, jnp.zeros((_N, 5), f32)], axis=1)

    ei = edge_index.reshape(2, _N, 2 * _K)
    src0 = ei[0, :, :_K].reshape(-1)
    dst0 = ei[1, :, :_K].reshape(-1)
    src1 = ei[0, :, ::2].reshape(-1)
    dst1 = ei[1, :, ::2].reshape(-1)
    srcidx = jnp.concatenate([src0, src1 + 2 * _N, src0 + _N, src1 + 3 * _N])
    dstidx = jnp.concatenate([dst0, dst1])
    zeros = jnp.zeros((_ZCH, _C), f32)

    nblk = _N // _BLK
    allpg, ed, t = pl.pallas_call(
        _prologue_body,
        grid=(nblk,),
        in_specs=[
            pl.BlockSpec((_BLK, _C), lambda i: (i, 0)),
            pl.BlockSpec((_BLK, 8), lambda i: (i, 0)),
            pl.BlockSpec((2, _C, _C), lambda i: (0, 0, 0)),
            pl.BlockSpec((2, _C, _C), lambda i: (0, 0, 0)),
            pl.BlockSpec((2, _C, _C), lambda i: (0, 0, 0)),
            pl.BlockSpec((2, 8, _C), lambda i: (0, 0, 0)),
            pl.BlockSpec((2, 8, _C), lambda i: (0, 0, 0)),
        ],
        out_specs=[
            pl.BlockSpec((4, _BLK, _C), lambda i: (0, i, 0)),
            pl.BlockSpec((2, _BLK, _C), lambda i: (0, i, 0)),
            pl.BlockSpec((2, _BLK, _C), lambda i: (0, i, 0)),
        ],
        out_shape=[
            jax.ShapeDtypeStruct((4, _N, _C), f32),
            jax.ShapeDtypeStruct((2, _N, _C), f32),
            jax.ShapeDtypeStruct((2, _N, _C), f32),
        ],
    )(x, posp, ws, wd, wl, wp, bp)

    sc = pl.kernel(
        _sc_body,
        out_type=jax.ShapeDtypeStruct((4 * _N, _C), f32),
        mesh=plsc.VectorSubcoreMesh(core_axis_name="c", subcore_axis_name="s"),
        scratch_types=[
            pltpu.VMEM((_CH,), jnp.int32),
            pltpu.VMEM((_CH,), jnp.int32),
            pltpu.VMEM((_CH, _C), f32),
            pltpu.VMEM_SHARED((_N, _C), f32),
            pltpu.SemaphoreType.DMA,
        ],
    )
    acc = sc(allpg.reshape(4 * _N, _C), srcidx, dstidx, zeros)

    out = pl.pallas_call(
        _epilogue_body,
        grid=(nblk,),
        in_specs=[
            pl.BlockSpec((4, _BLK, _C), lambda i: (0, i, 0)),
            pl.BlockSpec((2, _BLK, _C), lambda i: (0, i, 0)),
            pl.BlockSpec((2, _BLK, _C), lambda i: (0, i, 0)),
            pl.BlockSpec((_BLK, _C), lambda i: (i, 0)),
        ],
        out_specs=pl.BlockSpec((_BLK, _C), lambda i: (i, 0)),
        out_shape=jax.ShapeDtypeStruct((_N, _C), f32),
    )(acc.reshape(4, _N, _C), ed, t, x)
    return out
